# Initial kernel scaffold; baseline (speedup 1.0000x reference)
import functools

import jax
import jax.numpy as jnp
import numpy as np
from jax import lax
from jax.experimental import pallas as pl
from jax.experimental.pallas import tpu as pltpu
from jax.experimental.pallas import tpu_sc as plsc

GRID = 32
V = GRID ** 3
N_PTS = 262144
B, C = 2, 64
R = B * C
NC, NS = 2, 16
NW = NC * NS
ROWS_PER_W = R // NW
L = 16

PC = 2048
K = 2048
VS_EPS = np.float32(np.float32(0.1) + np.float32(1e-6))


def _body(pts_hbm, feat_hbm, out_hbm, seg_sh, acc, pts_buf, segc, seg_buf,
          fbuf_a, fbuf_b):
    c = lax.axis_index("c")
    s = lax.axis_index("s")
    wid = c * NS + s

    lane = lax.iota(jnp.int32, L)
    lane3 = lane * 3
    ones = jnp.ones((L,), jnp.float32)
    zeros = jnp.zeros((L,), jnp.float32)

    pts_per_sub = N_PTS // NS
    base_pt = s * pts_per_sub

    def p0_chunk(j, _):
        start = base_pt + j * PC
        pltpu.sync_copy(pts_hbm.at[pl.ds(start * 3, PC * 3)], pts_buf)

        def p0_vec(i, _):
            b = i * L
            gidx = lane3 + b * 3
            x = plsc.load_gather(pts_buf, [gidx])
            y = plsc.load_gather(pts_buf, [gidx + 1])
            z = plsc.load_gather(pts_buf, [gidx + 2])
            xi = (x / VS_EPS).astype(jnp.int32)
            yi = (y / VS_EPS).astype(jnp.int32)
            zi = (z / VS_EPS).astype(jnp.int32)
            segc[pl.ds(b, L)] = xi * (GRID * GRID) + yi * GRID + zi
            return 0

        lax.fori_loop(0, PC // L, p0_vec, 0)
        pltpu.sync_copy(segc, seg_sh.at[pl.ds(start, PC)])
        return 0

    lax.fori_loop(0, pts_per_sub // PC, p0_chunk, 0)
    plsc.subcore_barrier()

    for p in range(2):
        row0 = wid * ROWS_PER_W + 2 * p
        nzero = 3 * V if p == 0 else 2 * V

        def zbody(i, _):
            acc[pl.ds(i * L, L)] = zeros
            return 0

        lax.fori_loop(0, nzero // L, zbody, 0)

        def mchunk(j, _):
            off = j * K
            pltpu.sync_copy(seg_sh.at[pl.ds(off, K)], seg_buf)
            pltpu.sync_copy(feat_hbm.at[pl.ds(row0 * N_PTS + off, K)], fbuf_a)
            pltpu.sync_copy(feat_hbm.at[pl.ds((row0 + 1) * N_PTS + off, K)],
                            fbuf_b)

            def ibody(i, _):
                b = i * L
                seg = seg_buf[pl.ds(b, L)]
                va = fbuf_a[pl.ds(b, L)]
                vb = fbuf_b[pl.ds(b, L)]
                plsc.addupdate_scatter(acc, [seg], va)
                plsc.addupdate_scatter(acc, [seg + V], vb)
                if p == 0:
                    plsc.addupdate_scatter(acc, [seg + 2 * V], ones)
                return 0

            lax.fori_loop(0, K // L, ibody, 0)
            return 0

        lax.fori_loop(0, N_PTS // K, mchunk, 0)

        def fbody(i, _):
            b = i * L
            cnt = jnp.maximum(acc[pl.ds(2 * V + b, L)], 1.0)
            acc[pl.ds(b, L)] = acc[pl.ds(b, L)] / cnt
            acc[pl.ds(V + b, L)] = acc[pl.ds(V + b, L)] / cnt
            return 0

        lax.fori_loop(0, V // L, fbody, 0)
        pltpu.sync_copy(acc.at[pl.ds(0, V)], out_hbm.at[pl.ds(row0 * V, V)])
        pltpu.sync_copy(acc.at[pl.ds(V, V)],
                        out_hbm.at[pl.ds((row0 + 1) * V, V)])


@jax.jit
def _voxelize(pts_flat, feat_flat):
    mesh = plsc.VectorSubcoreMesh(core_axis_name="c", subcore_axis_name="s")
    return pl.kernel(
        _body,
        out_type=jax.ShapeDtypeStruct((R * V,), jnp.float32),
        mesh=mesh,
        scratch_types=[
            pltpu.VMEM_SHARED((N_PTS,), jnp.int32),
            pltpu.VMEM((3 * V,), jnp.float32),
            pltpu.VMEM((PC * 3,), jnp.float32),
            pltpu.VMEM((PC,), jnp.int32),
            pltpu.VMEM((K,), jnp.int32),
            pltpu.VMEM((K,), jnp.float32),
            pltpu.VMEM((K,), jnp.float32),
        ],
    )(pts_flat, feat_flat)


def kernel(points, features):
    out = _voxelize(points.reshape(-1), features.reshape(-1))
    return out.reshape(B, C, GRID, GRID, GRID)


# speedup vs baseline: 30.1169x; 30.1169x over previous
import functools

import jax
import jax.numpy as jnp
import numpy as np
from jax import lax
from jax.experimental import pallas as pl
from jax.experimental.pallas import tpu as pltpu
from jax.experimental.pallas import tpu_sc as plsc

GRID = 32
V = GRID ** 3
N_PTS = 262144
B, C = 2, 64
R = B * C
NC, NS = 2, 16
NW = NC * NS
ROWS_PER_W = R // NW
L = 16

PC = 2048
K = 2048
VS_EPS = np.float32(np.float32(0.1) + np.float32(1e-6))


def _body(pts_hbm, feat_hbm, out_hbm, seg_sh, acc, xbuf, ybuf, zbuf, segc,
          seg_buf, fbuf_a, fbuf_b):
    c = lax.axis_index("c")
    s = lax.axis_index("s")
    wid = c * NS + s

    ones = jnp.ones((L,), jnp.float32)
    zeros = jnp.zeros((L,), jnp.float32)

    pts_per_sub = N_PTS // NS
    base_pt = s * pts_per_sub

    def p0_chunk(j, _):
        start = base_pt + j * PC
        pltpu.sync_copy(pts_hbm.at[pl.ds(start, PC)], xbuf)
        pltpu.sync_copy(pts_hbm.at[pl.ds(N_PTS + start, PC)], ybuf)
        pltpu.sync_copy(pts_hbm.at[pl.ds(2 * N_PTS + start, PC)], zbuf)

        def p0_vec(i, _):
            b = i * L
            xi = (xbuf[pl.ds(b, L)] / VS_EPS).astype(jnp.int32)
            yi = (ybuf[pl.ds(b, L)] / VS_EPS).astype(jnp.int32)
            zi = (zbuf[pl.ds(b, L)] / VS_EPS).astype(jnp.int32)
            segc[pl.ds(b, L)] = xi * (GRID * GRID) + yi * GRID + zi
            return 0

        lax.fori_loop(0, PC // L, p0_vec, 0)
        pltpu.sync_copy(segc, seg_sh.at[pl.ds(start, PC)])
        return 0

    lax.fori_loop(0, pts_per_sub // PC, p0_chunk, 0)
    plsc.subcore_barrier()

    for p in range(2):
        row0 = wid * ROWS_PER_W + 2 * p
        nzero = 3 * V if p == 0 else 2 * V

        def zbody(i, _):
            acc[pl.ds(i * L, L)] = zeros
            return 0

        lax.fori_loop(0, nzero // L, zbody, 0)

        def mchunk(j, _):
            off = j * K
            pltpu.sync_copy(seg_sh.at[pl.ds(off, K)], seg_buf)
            pltpu.sync_copy(feat_hbm.at[pl.ds(row0 * N_PTS + off, K)], fbuf_a)
            pltpu.sync_copy(feat_hbm.at[pl.ds((row0 + 1) * N_PTS + off, K)],
                            fbuf_b)

            def ibody(i, _):
                b = i * L
                seg = seg_buf[pl.ds(b, L)]
                va = fbuf_a[pl.ds(b, L)]
                vb = fbuf_b[pl.ds(b, L)]
                plsc.addupdate_scatter(acc, [seg], va)
                plsc.addupdate_scatter(acc, [seg + V], vb)
                if p == 0:
                    plsc.addupdate_scatter(acc, [seg + 2 * V], ones)
                return 0

            lax.fori_loop(0, K // L, ibody, 0)
            return 0

        lax.fori_loop(0, N_PTS // K, mchunk, 0)

        def fbody(i, _):
            b = i * L
            cnt = jnp.maximum(acc[pl.ds(2 * V + b, L)], 1.0)
            acc[pl.ds(b, L)] = acc[pl.ds(b, L)] / cnt
            acc[pl.ds(V + b, L)] = acc[pl.ds(V + b, L)] / cnt
            return 0

        lax.fori_loop(0, V // L, fbody, 0)
        pltpu.sync_copy(acc.at[pl.ds(0, V)], out_hbm.at[pl.ds(row0 * V, V)])
        pltpu.sync_copy(acc.at[pl.ds(V, V)],
                        out_hbm.at[pl.ds((row0 + 1) * V, V)])


@jax.jit
def _voxelize(pts_flat, feat_flat):
    mesh = plsc.VectorSubcoreMesh(core_axis_name="c", subcore_axis_name="s")
    return pl.kernel(
        _body,
        out_type=jax.ShapeDtypeStruct((R * V,), jnp.float32),
        mesh=mesh,
        compiler_params=pltpu.CompilerParams(needs_layout_passes=False),
        scratch_types=[
            pltpu.VMEM_SHARED((N_PTS,), jnp.int32),
            pltpu.VMEM((3 * V,), jnp.float32),
            pltpu.VMEM((PC,), jnp.float32),
            pltpu.VMEM((PC,), jnp.float32),
            pltpu.VMEM((PC,), jnp.float32),
            pltpu.VMEM((PC,), jnp.int32),
            pltpu.VMEM((K,), jnp.int32),
            pltpu.VMEM((K,), jnp.float32),
            pltpu.VMEM((K,), jnp.float32),
        ],
    )(pts_flat, feat_flat)


def kernel(points, features):
    out = _voxelize(points.T.reshape(-1), features.reshape(-1))
    return out.reshape(B, C, GRID, GRID, GRID)


# speedup vs baseline: 44.9872x; 1.4938x over previous
import functools

import jax
import jax.numpy as jnp
import numpy as np
from jax import lax
from jax.experimental import pallas as pl
from jax.experimental.pallas import tpu as pltpu
from jax.experimental.pallas import tpu_sc as plsc

GRID = 32
V = GRID ** 3
N_PTS = 262144
B, C = 2, 64
R = B * C
NC, NS = 2, 16
NW = NC * NS
ROWS_PER_W = R // NW
L = 16

PC = 512
K = 2048
VS_EPS = np.float32(np.float32(0.1) + np.float32(1e-6))


def _body(pts_hbm, feat_hbm, out_hbm, seg_sh, acc, xbuf, ybuf, zbuf, segc,
          seg_buf, fbuf_a, fbuf_b, sseg0, sseg1, sfa0, sfa1, sfb0, sfb1):
    c = lax.axis_index("c")
    s = lax.axis_index("s")
    wid = c * NS + s

    ones = jnp.ones((L,), jnp.float32)
    zeros = jnp.zeros((L,), jnp.float32)

    pts_per_sub = N_PTS // NS
    base_pt = s * pts_per_sub

    def p0_chunk(j, _):
        start = base_pt + j * PC
        pltpu.sync_copy(pts_hbm.at[pl.ds(start, PC)], xbuf)
        pltpu.sync_copy(pts_hbm.at[pl.ds(N_PTS + start, PC)], ybuf)
        pltpu.sync_copy(pts_hbm.at[pl.ds(2 * N_PTS + start, PC)], zbuf)

        def p0_vec(i, _):
            b = i * L
            xi = (xbuf[pl.ds(b, L)] / VS_EPS).astype(jnp.int32)
            yi = (ybuf[pl.ds(b, L)] / VS_EPS).astype(jnp.int32)
            zi = (zbuf[pl.ds(b, L)] / VS_EPS).astype(jnp.int32)
            segc[pl.ds(b, L)] = xi * (GRID * GRID) + yi * GRID + zi
            return 0

        lax.fori_loop(0, PC // L, p0_vec, 0)
        pltpu.sync_copy(segc, seg_sh.at[pl.ds(start, PC)])
        return 0

    lax.fori_loop(0, pts_per_sub // PC, p0_chunk, 0)
    plsc.subcore_barrier()

    nch = N_PTS // K
    sems = ((sseg0, sfa0, sfb0), (sseg1, sfa1, sfb1))

    for p in range(2):
        row0 = wid * ROWS_PER_W + 2 * p
        nzero = 3 * V if p == 0 else 2 * V

        def zbody(i, _):
            acc[pl.ds(i * L, L)] = zeros
            return 0

        lax.fori_loop(0, nzero // L, zbody, 0)

        def issue(j, b):
            off = j * K
            ss, sa, sb = sems[b]
            pltpu.async_copy(seg_sh.at[pl.ds(off, K)], seg_buf.at[b], ss)
            pltpu.async_copy(feat_hbm.at[pl.ds(row0 * N_PTS + off, K)],
                             fbuf_a.at[b], sa)
            pltpu.async_copy(feat_hbm.at[pl.ds((row0 + 1) * N_PTS + off, K)],
                             fbuf_b.at[b], sb)

        def wait(b):
            ss, sa, sb = sems[b]
            pltpu.make_async_copy(seg_sh.at[pl.ds(0, K)], seg_buf.at[b],
                                  ss).wait()
            pltpu.make_async_copy(feat_hbm.at[pl.ds(0, K)], fbuf_a.at[b],
                                  sa).wait()
            pltpu.make_async_copy(feat_hbm.at[pl.ds(0, K)], fbuf_b.at[b],
                                  sb).wait()

        def compute(b, with_counts):
            def ibody(i, _):
                bb = i * L
                seg = seg_buf[b, pl.ds(bb, L)]
                va = fbuf_a[b, pl.ds(bb, L)]
                vb = fbuf_b[b, pl.ds(bb, L)]
                plsc.addupdate_scatter(acc, [seg], va)
                plsc.addupdate_scatter(acc, [seg + V], vb)
                if with_counts:
                    plsc.addupdate_scatter(acc, [seg + 2 * V], ones)
                return 0

            lax.fori_loop(0, K // L, ibody, 0)

        issue(0, 0)

        def mchunk(jj, _):
            for b in range(2):
                j = jj * 2 + b
                nxt = j + 1
                nxt = jnp.where(nxt >= nch, 0, nxt)
                issue(nxt, b ^ 1)
                wait(b)
                compute(b, p == 0)
            return 0

        lax.fori_loop(0, nch // 2, mchunk, 0)
        wait(0)

        def fbody(i, _):
            b = i * L
            cnt = jnp.maximum(acc[pl.ds(2 * V + b, L)], 1.0)
            acc[pl.ds(b, L)] = acc[pl.ds(b, L)] / cnt
            acc[pl.ds(V + b, L)] = acc[pl.ds(V + b, L)] / cnt
            return 0

        lax.fori_loop(0, V // L, fbody, 0)
        pltpu.sync_copy(acc.at[pl.ds(0, V)], out_hbm.at[pl.ds(row0 * V, V)])
        pltpu.sync_copy(acc.at[pl.ds(V, V)],
                        out_hbm.at[pl.ds((row0 + 1) * V, V)])


@jax.jit
def _voxelize(pts_flat, feat_flat):
    mesh = plsc.VectorSubcoreMesh(core_axis_name="c", subcore_axis_name="s")
    return pl.kernel(
        _body,
        out_type=jax.ShapeDtypeStruct((R * V,), jnp.float32),
        mesh=mesh,
        compiler_params=pltpu.CompilerParams(needs_layout_passes=False),
        scratch_types=[
            pltpu.VMEM_SHARED((N_PTS,), jnp.int32),
            pltpu.VMEM((3 * V,), jnp.float32),
            pltpu.VMEM((PC,), jnp.float32),
            pltpu.VMEM((PC,), jnp.float32),
            pltpu.VMEM((PC,), jnp.float32),
            pltpu.VMEM((PC,), jnp.int32),
            pltpu.VMEM((2, K), jnp.int32),
            pltpu.VMEM((2, K), jnp.float32),
            pltpu.VMEM((2, K), jnp.float32),
            pltpu.SemaphoreType.DMA,
            pltpu.SemaphoreType.DMA,
            pltpu.SemaphoreType.DMA,
            pltpu.SemaphoreType.DMA,
            pltpu.SemaphoreType.DMA,
            pltpu.SemaphoreType.DMA,
        ],
    )(pts_flat, feat_flat)


def kernel(points, features):
    out = _voxelize(points.T.reshape(-1), features.reshape(-1))
    return out.reshape(B, C, GRID, GRID, GRID)
